# Initial kernel scaffold; baseline (speedup 1.0000x reference)
import jax
import jax.numpy as jnp
from jax.experimental import pallas as pl


def kernel(x0, x1, x2, t_feat, conv_w, conv_b):
    raise NotImplementedError("write your pallas kernel here")


# speedup vs baseline: 4.5922x; 4.5922x over previous
import functools

import jax
import jax.numpy as jnp
from jax import lax
from jax.experimental import pallas as pl
from jax.experimental.pallas import tpu as pltpu
from jax.experimental.pallas import tpu_sc as plsc

N_CTX = 64000
WIN = 10
N_NODES = 10000
FEAT = 128
NB = 128
DROP = 0.5

NC = 2
NS = 16
NW = NC * NS
CPW = N_CTX // NW

CH = 40
NCHUNK = CPW // CH
ROWS = CH * WIN
GN = 5
GR = ROWS // GN

R = 80
NCHS = CPW // R
NSEG_PAD = 10240
NPASS = 2
NSEG_H = NSEG_PAD // NPASS
ACC_ROWS = NSEG_H + NS * 8
WPS = NSEG_H // NS
ZB = 64

_MESH = plsc.VectorSubcoreMesh(
    core_axis_name="c", subcore_axis_name="s", num_cores=NC, num_subcores=NS
)


def _proj_body(tf_ref, w2_ref, bias_ref, out_ref):
    acc = lax.dot_general(
        tf_ref[...], w2_ref[...], (((1,), (0,)), ((), ())),
        preferred_element_type=jnp.float32,
        precision=lax.Precision.HIGHEST,
    )
    out_ref[...] = acc * (1.0 - DROP) + bias_ref[...]


def _project(t_feat, w2, bias_full):
    mb = 400
    return pl.pallas_call(
        _proj_body,
        grid=(N_NODES // mb,),
        in_specs=[
            pl.BlockSpec((mb, FEAT), lambda i: (i, 0)),
            pl.BlockSpec((FEAT, WIN * NB), lambda i: (0, 0)),
            pl.BlockSpec((1, WIN * NB), lambda i: (0, 0)),
        ],
        out_specs=pl.BlockSpec((mb, WIN * NB), lambda i: (i, 0)),
        out_shape=jax.ShapeDtypeStruct((N_NODES, WIN * NB), jnp.float32),
    )(t_feat, w2, bias_full)


@functools.partial(
    pl.kernel,
    out_type=jax.ShapeDtypeStruct((N_CTX, NB), jnp.float32),
    mesh=_MESH,
    scratch_types=[
        pltpu.VMEM((ROWS,), jnp.int32),
        pltpu.VMEM((GN, GR), jnp.int32),
        pltpu.VMEM((ROWS, NB), jnp.float32),
        pltpu.VMEM((CH, NB), jnp.float32),
        pltpu.SemaphoreType.DMA,
    ],
)
def _winenc(p2_hbm, x0_hbm, out_hbm, x0c, idxc, rows, outc, sem):
    wid = lax.axis_index("s") * NC + lax.axis_index("c")
    lanes = lax.iota(jnp.int32, 16)

    def chunk_body(c, carry):
        base_ctx = wid * CPW + c * CH
        base_j = base_ctx * WIN
        pltpu.sync_copy(x0_hbm.at[pl.ds(base_j, ROWS)], x0c)
        for k in range(ROWS // 16):
            x = x0c[pl.ds(k * 16, 16)]
            wpos = lax.rem(lanes + (k * 16) % WIN, WIN)
            idxc[k // GN, pl.ds((k % GN) * 16, 16)] = x * WIN + wpos
        cps = [
            pltpu.async_copy(
                p2_hbm.at[idxc.at[g]], rows.at[pl.ds(g * GR, GR)], sem
            )
            for g in range(GN)
        ]
        for cp in cps:
            cp.wait()

        def ctx_body(b, carry2):
            r0 = b * WIN
            for h in range(NB // 16):
                sl = pl.ds(h * 16, 16)
                acc = rows[r0, sl]
                for w in range(1, WIN):
                    acc = acc + rows[r0 + w, sl]
                outc[b, sl] = acc
            return carry2

        lax.fori_loop(0, CH, ctx_body, 0)
        pltpu.sync_copy(outc, out_hbm.at[pl.ds(base_ctx, CH)])
        return carry

    lax.fori_loop(0, NCHUNK, chunk_body, 0)


@functools.partial(
    pl.kernel,
    out_type=(
        jax.ShapeDtypeStruct((NC, NSEG_PAD, NB), jnp.float32),
        jax.ShapeDtypeStruct((NC, NSEG_PAD, NB), jnp.float32),
    ),
    mesh=_MESH,
    scratch_types=[
        pltpu.VMEM((R, NB), jnp.float32),
        pltpu.VMEM((1, R), jnp.int32),
        pltpu.VMEM((R, NB), jnp.float32),
        pltpu.VMEM((ZB, NB), jnp.float32),
        pltpu.VMEM_SHARED((ACC_ROWS, NB), jnp.float32),
    ],
)
def _segsum(win_hbm, x1_hbm, fp_hbm, cp_hbm, rowc, labc, ones, zb, acc):
    cid = lax.axis_index("c")
    sid = lax.axis_index("s")
    wid = sid * NC + cid
    zero16 = jnp.zeros((16,), jnp.float32)
    one16 = jnp.ones((16,), jnp.float32)

    def fill_body(i, carry):
        for h in range(NB // 16):
            zb[i, pl.ds(h * 16, 16)] = zero16
        return carry

    lax.fori_loop(0, ZB, fill_body, 0)

    def ones_body(i, carry):
        for h in range(NB // 16):
            ones[i, pl.ds(h * 16, 16)] = one16
        return carry

    lax.fori_loop(0, R, ones_body, 0)

    dump0 = NSEG_H + sid * 8

    for kind in range(2):
        src_is_rows = kind == 0
        out_hbm = fp_hbm if src_is_rows else cp_hbm
        for p in range(NPASS):
            for i in range(WPS // ZB):
                pltpu.sync_copy(zb, acc.at[pl.ds(sid * WPS + i * ZB, ZB)])
            pltpu.sync_copy(zb.at[pl.ds(0, 8)], acc.at[pl.ds(dump0, 8)])
            plsc.subcore_barrier()

            def chunk_body(c, carry):
                gc = wid * NCHS + c
                pltpu.sync_copy(x1_hbm.at[pl.ds(gc * R, R)], labc.at[0])
                for k in range(R // 16):
                    sl = pl.ds(k * 16, 16)
                    idx = labc[0, sl] - p * NSEG_H
                    ok = (idx >= 0) & (idx < NSEG_H)
                    labc[0, sl] = jnp.where(ok, idx, dump0)
                if src_is_rows:
                    pltpu.sync_copy(win_hbm.at[pl.ds(gc * R, R)], rowc)
                    pltpu.sync_copy(rowc, acc.at[labc.at[0]], add=True)
                else:
                    pltpu.sync_copy(ones, acc.at[labc.at[0]], add=True)
                return carry

            lax.fori_loop(0, NCHS, chunk_body, 0)
            plsc.subcore_barrier()

            pltpu.sync_copy(
                acc.at[pl.ds(sid * WPS, WPS)],
                out_hbm.at[cid, pl.ds(p * NSEG_H + sid * WPS, WPS)],
            )
            plsc.subcore_barrier()


def _avg_body(fp_ref, cp_ref, out_ref):
    s = fp_ref[0] + fp_ref[1]
    c = cp_ref[0, :, 0:1] + cp_ref[1, :, 0:1]
    out_ref[...] = s / c


def _combine(fp, cp):
    mb = 512
    return pl.pallas_call(
        _avg_body,
        grid=(NSEG_PAD // mb,),
        in_specs=[
            pl.BlockSpec((NC, mb, NB), lambda i: (0, i, 0)),
            pl.BlockSpec((NC, mb, NB), lambda i: (0, i, 0)),
        ],
        out_specs=pl.BlockSpec((mb, NB), lambda i: (i, 0)),
        out_shape=jax.ShapeDtypeStruct((NSEG_PAD, NB), jnp.float32),
    )(fp, cp)


def kernel(x0, x1, x2, t_feat, conv_w, conv_b):
    x0f = x0.astype(jnp.int32).reshape(-1)
    x1i = x1.astype(jnp.int32)
    w2 = jnp.transpose(conv_w, (1, 2, 0)).reshape(FEAT, WIN * NB)
    bias_full = jnp.concatenate(
        [conv_b, jnp.zeros(((WIN - 1) * NB,), jnp.float32)]
    ).reshape(1, WIN * NB)

    p = _project(t_feat, w2, bias_full)
    p2 = p.reshape(N_NODES * WIN, NB)
    win_enc = _winenc(p2, x0f)
    fp, cp = _segsum(win_enc, x1i)
    favg_pad = _combine(fp, cp)
    return (win_enc, favg_pad[:N_NODES])


# speedup vs baseline: 4.6076x; 1.0034x over previous
import functools

import jax
import jax.numpy as jnp
from jax import lax
from jax.experimental import pallas as pl
from jax.experimental.pallas import tpu as pltpu
from jax.experimental.pallas import tpu_sc as plsc

N_CTX = 64000
WIN = 10
N_NODES = 10000
FEAT = 128
NB = 128
DROP = 0.5

NC = 2
NS = 16
NW = NC * NS
CPW = N_CTX // NW

CH = 40
NCHUNK = CPW // CH
ROWS = CH * WIN
GN = 5
GR = ROWS // GN

R = 80
NCHS = CPW // R
NSEG_PAD = 10240
NPASS = 2
NSEG_H = NSEG_PAD // NPASS
ACC_ROWS = NSEG_H + NS * 8
WPS = NSEG_H // NS
ZB = 64

_MESH = plsc.VectorSubcoreMesh(
    core_axis_name="c", subcore_axis_name="s", num_cores=NC, num_subcores=NS
)


def _proj_body(tf_ref, w2_ref, bias_ref, out_ref):
    acc = lax.dot_general(
        tf_ref[...], w2_ref[...], (((1,), (0,)), ((), ())),
        preferred_element_type=jnp.float32,
        precision=lax.Precision.HIGHEST,
    )
    out_ref[...] = acc * (1.0 - DROP) + bias_ref[...]


def _project(t_feat, w2, bias_full):
    mb = 400
    return pl.pallas_call(
        _proj_body,
        grid=(N_NODES // mb,),
        in_specs=[
            pl.BlockSpec((mb, FEAT), lambda i: (i, 0)),
            pl.BlockSpec((FEAT, WIN * NB), lambda i: (0, 0)),
            pl.BlockSpec((1, WIN * NB), lambda i: (0, 0)),
        ],
        out_specs=pl.BlockSpec((mb, WIN * NB), lambda i: (i, 0)),
        out_shape=jax.ShapeDtypeStruct((N_NODES, WIN * NB), jnp.float32),
    )(t_feat, w2, bias_full)


@functools.partial(
    pl.kernel,
    out_type=jax.ShapeDtypeStruct((N_CTX, NB), jnp.float32),
    mesh=_MESH,
    scratch_types=[
        pltpu.VMEM((ROWS,), jnp.int32),
        pltpu.VMEM((GN, GR), jnp.int32),
        pltpu.VMEM((ROWS, NB), jnp.float32),
        pltpu.VMEM((CH, NB), jnp.float32),
        pltpu.SemaphoreType.DMA,
    ],
)
def _winenc(p2_hbm, x0_hbm, out_hbm, x0c, idxc, rows, outc, sem):
    wid = lax.axis_index("s") * NC + lax.axis_index("c")
    lanes = lax.iota(jnp.int32, 16)

    def chunk_body(c, carry):
        base_ctx = wid * CPW + c * CH
        base_j = base_ctx * WIN
        pltpu.sync_copy(x0_hbm.at[pl.ds(base_j, ROWS)], x0c)
        for k in range(ROWS // 16):
            x = x0c[pl.ds(k * 16, 16)]
            wpos = lax.rem(lanes + (k * 16) % WIN, WIN)
            idxc[k // GN, pl.ds((k % GN) * 16, 16)] = x * WIN + wpos
        cps = [
            pltpu.async_copy(
                p2_hbm.at[idxc.at[g]], rows.at[pl.ds(g * GR, GR)], sem
            )
            for g in range(GN)
        ]
        for cp in cps:
            cp.wait()

        def ctx_body(b, carry2):
            r0 = b * WIN
            for h in range(NB // 16):
                sl = pl.ds(h * 16, 16)
                acc = rows[r0, sl]
                for w in range(1, WIN):
                    acc = acc + rows[r0 + w, sl]
                outc[b, sl] = acc
            return carry2

        lax.fori_loop(0, CH, ctx_body, 0)
        pltpu.sync_copy(outc, out_hbm.at[pl.ds(base_ctx, CH)])
        return carry

    lax.fori_loop(0, NCHUNK, chunk_body, 0)


@functools.partial(
    pl.kernel,
    out_type=(
        jax.ShapeDtypeStruct((NC, NSEG_PAD, NB), jnp.float32),
        jax.ShapeDtypeStruct((NC, NSEG_PAD, NB), jnp.float32),
    ),
    mesh=_MESH,
    scratch_types=[
        pltpu.VMEM((R, NB), jnp.float32),
        pltpu.VMEM((1, R), jnp.int32),
        pltpu.VMEM((R, NB), jnp.float32),
        pltpu.VMEM((ZB, NB), jnp.float32),
        pltpu.VMEM_SHARED((ACC_ROWS, NB), jnp.float32),
    ],
)
def _segsum(win_hbm, x1_hbm, fp_hbm, cp_hbm, rowc, labc, ones, zb, acc):
    cid = lax.axis_index("c")
    sid = lax.axis_index("s")
    wid = sid * NC + cid
    zero16 = jnp.zeros((16,), jnp.float32)
    one16 = jnp.ones((16,), jnp.float32)

    def fill_body(i, carry):
        for h in range(NB // 16):
            zb[i, pl.ds(h * 16, 16)] = zero16
        return carry

    lax.fori_loop(0, ZB, fill_body, 0)

    def ones_body(i, carry):
        for h in range(NB // 16):
            ones[i, pl.ds(h * 16, 16)] = one16
        return carry

    lax.fori_loop(0, R, ones_body, 0)

    dump0 = NSEG_H + sid * 8

    for kind in range(2):
        src_is_rows = kind == 0
        out_hbm = fp_hbm if src_is_rows else cp_hbm
        for p in range(NPASS):
            for i in range(WPS // ZB):
                pltpu.sync_copy(zb, acc.at[pl.ds(sid * WPS + i * ZB, ZB)])
            pltpu.sync_copy(zb.at[pl.ds(0, 8)], acc.at[pl.ds(dump0, 8)])
            plsc.subcore_barrier()

            def chunk_body(c, carry):
                gc = wid * NCHS + c
                pltpu.sync_copy(x1_hbm.at[pl.ds(gc * R, R)], labc.at[0])
                minv = labc[0, pl.ds(0, 16)][0]
                maxv = labc[0, pl.ds(R - 16, 16)][15]
                inwin = (maxv >= p * NSEG_H) & (minv < (p + 1) * NSEG_H)

                @pl.when(inwin)
                def _do_chunk():
                    for k in range(R // 16):
                        sl = pl.ds(k * 16, 16)
                        idx = labc[0, sl] - p * NSEG_H
                        ok = (idx >= 0) & (idx < NSEG_H)
                        labc[0, sl] = jnp.where(ok, idx, dump0)
                    if src_is_rows:
                        pltpu.sync_copy(win_hbm.at[pl.ds(gc * R, R)], rowc)
                        pltpu.sync_copy(rowc, acc.at[labc.at[0]], add=True)
                    else:
                        pltpu.sync_copy(ones, acc.at[labc.at[0]], add=True)

                return carry

            lax.fori_loop(0, NCHS, chunk_body, 0)
            plsc.subcore_barrier()

            pltpu.sync_copy(
                acc.at[pl.ds(sid * WPS, WPS)],
                out_hbm.at[cid, pl.ds(p * NSEG_H + sid * WPS, WPS)],
            )
            plsc.subcore_barrier()


def _avg_body(fp_ref, cp_ref, out_ref):
    s = fp_ref[0] + fp_ref[1]
    c = cp_ref[0, :, 0:1] + cp_ref[1, :, 0:1]
    out_ref[...] = s / c


def _combine(fp, cp):
    mb = 512
    return pl.pallas_call(
        _avg_body,
        grid=(NSEG_PAD // mb,),
        in_specs=[
            pl.BlockSpec((NC, mb, NB), lambda i: (0, i, 0)),
            pl.BlockSpec((NC, mb, NB), lambda i: (0, i, 0)),
        ],
        out_specs=pl.BlockSpec((mb, NB), lambda i: (i, 0)),
        out_shape=jax.ShapeDtypeStruct((NSEG_PAD, NB), jnp.float32),
    )(fp, cp)


def kernel(x0, x1, x2, t_feat, conv_w, conv_b):
    x0f = x0.astype(jnp.int32).reshape(-1)
    x1i = x1.astype(jnp.int32)
    w2 = jnp.transpose(conv_w, (1, 2, 0)).reshape(FEAT, WIN * NB)
    bias_full = jnp.concatenate(
        [conv_b, jnp.zeros(((WIN - 1) * NB,), jnp.float32)]
    ).reshape(1, WIN * NB)

    p = _project(t_feat, w2, bias_full)
    p2 = p.reshape(N_NODES * WIN, NB)
    win_enc = _winenc(p2, x0f)
    fp, cp = _segsum(win_enc, x1i)
    favg_pad = _combine(fp, cp)
    return (win_enc, favg_pad[:N_NODES])


# speedup vs baseline: 5.3684x; 1.1651x over previous
import functools

import jax
import jax.numpy as jnp
from jax import lax
from jax.experimental import pallas as pl
from jax.experimental.pallas import tpu as pltpu
from jax.experimental.pallas import tpu_sc as plsc

N_CTX = 64000
WIN = 10
N_NODES = 10000
FEAT = 128
NB = 128
DROP = 0.5

NC = 2
NS = 16
NW = NC * NS
CPW = N_CTX // NW

CH = 40
NCHUNK = CPW // CH
ROWS = CH * WIN
GN = 5
GR = ROWS // GN

R = 80
NCHS = CPW // R
NSEG_PAD = 10240
NPASS = 1
NSEG_H = NSEG_PAD // NPASS
ACC_ROWS = NSEG_H + NS * 8
WPS = NSEG_H // NS
ZB = 64

_MESH = plsc.VectorSubcoreMesh(
    core_axis_name="c", subcore_axis_name="s", num_cores=NC, num_subcores=NS
)


def _proj_body(tf_ref, w2_ref, bias_ref, out_ref):
    acc = lax.dot_general(
        tf_ref[...], w2_ref[...], (((1,), (0,)), ((), ())),
        preferred_element_type=jnp.float32,
        precision=lax.Precision.HIGHEST,
    )
    out_ref[...] = acc * (1.0 - DROP) + bias_ref[...]


def _project(t_feat, w2, bias_full):
    mb = 400
    return pl.pallas_call(
        _proj_body,
        grid=(N_NODES // mb,),
        in_specs=[
            pl.BlockSpec((mb, FEAT), lambda i: (i, 0)),
            pl.BlockSpec((FEAT, WIN * NB), lambda i: (0, 0)),
            pl.BlockSpec((1, WIN * NB), lambda i: (0, 0)),
        ],
        out_specs=pl.BlockSpec((mb, WIN * NB), lambda i: (i, 0)),
        out_shape=jax.ShapeDtypeStruct((N_NODES, WIN * NB), jnp.float32),
    )(t_feat, w2, bias_full)


@functools.partial(
    pl.kernel,
    out_type=jax.ShapeDtypeStruct((N_CTX, NB), jnp.float32),
    mesh=_MESH,
    scratch_types=[
        pltpu.VMEM((ROWS,), jnp.int32),
        pltpu.VMEM((GN, GR), jnp.int32),
        pltpu.VMEM((ROWS, NB), jnp.float32),
        pltpu.VMEM((CH, NB), jnp.float32),
        pltpu.SemaphoreType.DMA,
    ],
)
def _winenc(p2_hbm, x0_hbm, out_hbm, x0c, idxc, rows, outc, sem):
    wid = lax.axis_index("s") * NC + lax.axis_index("c")
    lanes = lax.iota(jnp.int32, 16)

    def chunk_body(c, carry):
        base_ctx = wid * CPW + c * CH
        base_j = base_ctx * WIN
        pltpu.sync_copy(x0_hbm.at[pl.ds(base_j, ROWS)], x0c)
        for k in range(ROWS // 16):
            x = x0c[pl.ds(k * 16, 16)]
            wpos = lax.rem(lanes + (k * 16) % WIN, WIN)
            idxc[k // GN, pl.ds((k % GN) * 16, 16)] = x * WIN + wpos
        cps = [
            pltpu.async_copy(
                p2_hbm.at[idxc.at[g]], rows.at[pl.ds(g * GR, GR)], sem
            )
            for g in range(GN)
        ]
        for cp in cps:
            cp.wait()

        def ctx_body(b, carry2):
            r0 = b * WIN
            for h in range(NB // 16):
                sl = pl.ds(h * 16, 16)
                acc = rows[r0, sl]
                for w in range(1, WIN):
                    acc = acc + rows[r0 + w, sl]
                outc[b, sl] = acc
            return carry2

        lax.fori_loop(0, CH, ctx_body, 0)
        pltpu.sync_copy(outc, out_hbm.at[pl.ds(base_ctx, CH)])
        return carry

    lax.fori_loop(0, NCHUNK, chunk_body, 0)


R = 80
NCHS = CPW // R
GCH = 1
NG = NCHS
GROWS = R
NSEG_PAD = 10240
WPS = NSEG_PAD // NS
ZB = 64


@functools.partial(
    pl.kernel,
    out_type=(
        jax.ShapeDtypeStruct((NC, NSEG_PAD, NB), jnp.float32),
        jax.ShapeDtypeStruct((NC, NSEG_PAD, NB), jnp.float32),
    ),
    mesh=_MESH,
    scratch_types=[
        pltpu.VMEM((2, GROWS, NB), jnp.float32),
        pltpu.VMEM((NCHS, R), jnp.int32),
        pltpu.VMEM((R, NB), jnp.float32),
        pltpu.VMEM((ZB, NB), jnp.float32),
        pltpu.VMEM_SHARED((NSEG_PAD, NB), jnp.float32),
        pltpu.SemaphoreType.DMA,
        pltpu.SemaphoreType.DMA,
        pltpu.SemaphoreType.DMA,
        pltpu.SemaphoreType.DMA,
    ],
)
def _segsum(win_hbm, x1g_hbm, fp_hbm, cp_hbm, rowg, labs, ones, zb,
            acc, rs0, rs1, ss0, ss1):
    cid = lax.axis_index("c")
    sid = lax.axis_index("s")
    wid = sid * NC + cid
    zero16 = jnp.zeros((16,), jnp.float32)
    one16 = jnp.ones((16,), jnp.float32)
    rsem = (rs0, rs1)
    ssem = (ss0, ss1)

    def fill_body(i, carry):
        for h in range(NB // 16):
            zb[i, pl.ds(h * 16, 16)] = zero16
            ones[i, pl.ds(h * 16, 16)] = one16
        return carry

    lax.fori_loop(0, ZB, fill_body, 0)

    def ones_body(i, carry):
        for h in range(NB // 16):
            ones[ZB + i, pl.ds(h * 16, 16)] = one16
        return carry

    lax.fori_loop(0, R - ZB, ones_body, 0)

    pltpu.sync_copy(x1g_hbm.at[wid], labs)
    for i in range(WPS // ZB):
        pltpu.sync_copy(zb, acc.at[pl.ds(sid * WPS + i * ZB, ZB)])
    plsc.subcore_barrier()

    row0 = wid * CPW
    pend = [None, None]
    rcps = [None, None]
    rcps[0] = pltpu.async_copy(
        win_hbm.at[pl.ds(row0, R)], rowg.at[0], rsem[0]
    )
    for c in range(NCHS):
        bi = c % 2
        if pend[1 - bi] is not None:
            pend[1 - bi].wait()
            pend[1 - bi] = None
        if c + 1 < NCHS:
            rcps[1 - bi] = pltpu.async_copy(
                win_hbm.at[pl.ds(row0 + (c + 1) * R, R)],
                rowg.at[1 - bi], rsem[1 - bi],
            )
        rcps[bi].wait()
        pend[bi] = pltpu.async_copy(
            rowg.at[bi], acc.at[labs.at[c]], ssem[bi], add=True,
        )
    for bi in range(2):
        if pend[bi] is not None:
            pend[bi].wait()
    plsc.subcore_barrier()

    pltpu.sync_copy(
        acc.at[pl.ds(sid * WPS, WPS)], fp_hbm.at[cid, pl.ds(sid * WPS, WPS)]
    )
    plsc.subcore_barrier()

    for i in range(WPS // ZB):
        pltpu.sync_copy(zb, acc.at[pl.ds(sid * WPS + i * ZB, ZB)])
    plsc.subcore_barrier()
    cdescs = []
    for c in range(NCHS):
        cdescs.append(pltpu.async_copy(
            ones, acc.at[labs.at[c]], ssem[c % 2], add=True,
        ))
    for d in cdescs:
        d.wait()
    plsc.subcore_barrier()
    pltpu.sync_copy(
        acc.at[pl.ds(sid * WPS, WPS)], cp_hbm.at[cid, pl.ds(sid * WPS, WPS)]
    )


def _avg_body(fp_ref, cp_ref, out_ref):
    s = fp_ref[0] + fp_ref[1]
    c = cp_ref[0, :, 0:1] + cp_ref[1, :, 0:1]
    out_ref[...] = s / c


def _combine(fp, cp):
    mb = 512
    return pl.pallas_call(
        _avg_body,
        grid=(NSEG_PAD // mb,),
        in_specs=[
            pl.BlockSpec((NC, mb, NB), lambda i: (0, i, 0)),
            pl.BlockSpec((NC, mb, NB), lambda i: (0, i, 0)),
        ],
        out_specs=pl.BlockSpec((mb, NB), lambda i: (i, 0)),
        out_shape=jax.ShapeDtypeStruct((NSEG_PAD, NB), jnp.float32),
    )(fp, cp)


def kernel(x0, x1, x2, t_feat, conv_w, conv_b):
    x0f = x0.astype(jnp.int32).reshape(-1)
    x1g = x1.astype(jnp.int32).reshape(NW, NCHS, R)
    w2 = jnp.transpose(conv_w, (1, 2, 0)).reshape(FEAT, WIN * NB)
    bias_full = jnp.concatenate(
        [conv_b, jnp.zeros(((WIN - 1) * NB,), jnp.float32)]
    ).reshape(1, WIN * NB)

    p = _project(t_feat, w2, bias_full)
    p2 = p.reshape(N_NODES * WIN, NB)
    win_enc = _winenc(p2, x0f)
    fp, cp = _segsum(win_enc, x1g)
    favg_pad = _combine(fp, cp)
    return (win_enc, favg_pad[:N_NODES])


# speedup vs baseline: 6.6882x; 1.2459x over previous
import functools

import jax
import jax.numpy as jnp
from jax import lax
from jax.experimental import pallas as pl
from jax.experimental.pallas import tpu as pltpu
from jax.experimental.pallas import tpu_sc as plsc

N_CTX = 64000
WIN = 10
N_NODES = 10000
FEAT = 128
NB = 128
DROP = 0.5

NC = 2
NS = 16
NW = NC * NS
CPW = N_CTX // NW

CH = 40
NCHUNK = CPW // CH
ROWS = CH * WIN
GN = 5
GR = ROWS // GN

R = 80
NCHS = CPW // R
NSEG_PAD = 10240
NPASS = 1
NSEG_H = NSEG_PAD // NPASS
ACC_ROWS = NSEG_H + NS * 8
WPS = NSEG_H // NS
ZB = 64

_MESH = plsc.VectorSubcoreMesh(
    core_axis_name="c", subcore_axis_name="s", num_cores=NC, num_subcores=NS
)


def _proj_body(tf_ref, w2_ref, bias_ref, out_ref):
    acc = lax.dot_general(
        tf_ref[...], w2_ref[...], (((1,), (0,)), ((), ())),
        preferred_element_type=jnp.float32,
        precision=lax.Precision.HIGHEST,
    )
    out_ref[...] = acc * (1.0 - DROP) + bias_ref[...]


def _project(t_feat, w2, bias_full):
    mb = 400
    return pl.pallas_call(
        _proj_body,
        grid=(N_NODES // mb,),
        in_specs=[
            pl.BlockSpec((mb, FEAT), lambda i: (i, 0)),
            pl.BlockSpec((FEAT, WIN * NB), lambda i: (0, 0)),
            pl.BlockSpec((1, WIN * NB), lambda i: (0, 0)),
        ],
        out_specs=pl.BlockSpec((mb, WIN * NB), lambda i: (i, 0)),
        out_shape=jax.ShapeDtypeStruct((N_NODES, WIN * NB), jnp.float32),
    )(t_feat, w2, bias_full)


@functools.partial(
    pl.kernel,
    out_type=jax.ShapeDtypeStruct((N_CTX, NB), jnp.float32),
    mesh=_MESH,
    scratch_types=[
        pltpu.VMEM((ROWS,), jnp.int32),
        pltpu.VMEM((ROWS,), jnp.int32),
        pltpu.VMEM((GN, GR), jnp.int32),
        pltpu.VMEM((GN, GR), jnp.int32),
        pltpu.VMEM((ROWS, NB), jnp.float32),
        pltpu.VMEM((ROWS, NB), jnp.float32),
        pltpu.VMEM((CH, NB), jnp.float32),
        pltpu.SemaphoreType.DMA,
        pltpu.SemaphoreType.DMA,
    ],
)
def _winenc(p2_hbm, x0_hbm, out_hbm, x0c0, x0c1, idxc0, idxc1,
            rows0, rows1, outc, gs0, gs1):
    wid = lax.axis_index("s") * NC + lax.axis_index("c")
    lanes = lax.iota(jnp.int32, 16)
    gsem = (gs0, gs1)
    x0cs = (x0c0, x0c1)
    idxcs = (idxc0, idxc1)
    rowss = (rows0, rows1)

    def _prep(c, bi):
        x0c, idxc, rows = x0cs[bi], idxcs[bi], rowss[bi]
        base_j = (wid * CPW + c * CH) * WIN
        pltpu.sync_copy(x0_hbm.at[pl.ds(base_j, ROWS)], x0c)
        for k in range(ROWS // 16):
            x = x0c[pl.ds(k * 16, 16)]
            wpos = lax.rem(lanes + (k * 16) % WIN, WIN)
            idxc[k // GN, pl.ds((k % GN) * 16, 16)] = x * WIN + wpos
        for g in range(GN):
            pltpu.async_copy(
                p2_hbm.at[idxc.at[g]],
                rows.at[pl.ds(g * GR, GR)], gsem[bi],
            )

    def _drain(bi):
        pltpu.make_async_copy(
            p2_hbm.at[pl.ds(0, ROWS)], rowss[bi], gsem[bi]
        ).wait()

    def _compute(c, bi):
        rows = rowss[bi]

        def ctx_body(b, carry2):
            r0 = b * WIN
            for h in range(NB // 16):
                sl = pl.ds(h * 16, 16)
                acc = rows[r0, sl]
                for w in range(1, WIN):
                    acc = acc + rows[r0 + w, sl]
                outc[b, sl] = acc
            return carry2

        lax.fori_loop(0, CH, ctx_body, 0)
        pltpu.sync_copy(outc, out_hbm.at[pl.ds(wid * CPW + c * CH, CH)])

    _prep(0, 0)

    def pair_body(p, carry):
        c0 = 2 * p
        _prep(c0 + 1, 1)
        _drain(0)
        _compute(c0, 0)
        _prep(lax.rem(c0 + 2, NCHUNK), 0)
        _drain(1)
        _compute(c0 + 1, 1)
        return carry

    lax.fori_loop(0, NCHUNK // 2, pair_body, 0)
    _drain(0)


R = 80
NCHS = CPW // R
GCH = 1
NG = NCHS
GROWS = R
NSEG_PAD = 10240
WPS = NSEG_PAD // NS
ZB = 64


@functools.partial(
    pl.kernel,
    out_type=(
        jax.ShapeDtypeStruct((NC, NSEG_PAD, NB), jnp.float32),
        jax.ShapeDtypeStruct((NC, NSEG_PAD, NB), jnp.float32),
    ),
    mesh=_MESH,
    scratch_types=[
        pltpu.VMEM((2, GROWS, NB), jnp.float32),
        pltpu.VMEM((NCHS, R), jnp.int32),
        pltpu.VMEM((R, NB), jnp.float32),
        pltpu.VMEM((ZB, NB), jnp.float32),
        pltpu.VMEM_SHARED((NSEG_PAD, NB), jnp.float32),
        pltpu.SemaphoreType.DMA,
        pltpu.SemaphoreType.DMA,
        pltpu.SemaphoreType.DMA,
        pltpu.SemaphoreType.DMA,
    ],
)
def _segsum(win_hbm, x1g_hbm, fp_hbm, cp_hbm, rowg, labs, ones, zb,
            acc, rs0, rs1, ss0, ss1):
    cid = lax.axis_index("c")
    sid = lax.axis_index("s")
    wid = sid * NC + cid
    zero16 = jnp.zeros((16,), jnp.float32)
    one16 = jnp.ones((16,), jnp.float32)
    rsem = (rs0, rs1)
    ssem = (ss0, ss1)

    def fill_body(i, carry):
        for h in range(NB // 16):
            zb[i, pl.ds(h * 16, 16)] = zero16
            ones[i, pl.ds(h * 16, 16)] = one16
        return carry

    lax.fori_loop(0, ZB, fill_body, 0)

    def ones_body(i, carry):
        for h in range(NB // 16):
            ones[ZB + i, pl.ds(h * 16, 16)] = one16
        return carry

    lax.fori_loop(0, R - ZB, ones_body, 0)

    pltpu.sync_copy(x1g_hbm.at[wid], labs)
    for i in range(WPS // ZB):
        pltpu.sync_copy(zb, acc.at[pl.ds(sid * WPS + i * ZB, ZB)])
    plsc.subcore_barrier()

    row0 = wid * CPW
    pend = [None, None]
    rcps = [None, None]
    rcps[0] = pltpu.async_copy(
        win_hbm.at[pl.ds(row0, R)], rowg.at[0], rsem[0]
    )
    for c in range(NCHS):
        bi = c % 2
        if pend[1 - bi] is not None:
            pend[1 - bi].wait()
            pend[1 - bi] = None
        if c + 1 < NCHS:
            rcps[1 - bi] = pltpu.async_copy(
                win_hbm.at[pl.ds(row0 + (c + 1) * R, R)],
                rowg.at[1 - bi], rsem[1 - bi],
            )
        rcps[bi].wait()
        pend[bi] = pltpu.async_copy(
            rowg.at[bi], acc.at[labs.at[c]], ssem[bi], add=True,
        )
    for bi in range(2):
        if pend[bi] is not None:
            pend[bi].wait()
    plsc.subcore_barrier()

    pltpu.sync_copy(
        acc.at[pl.ds(sid * WPS, WPS)], fp_hbm.at[cid, pl.ds(sid * WPS, WPS)]
    )
    plsc.subcore_barrier()

    for i in range(WPS // ZB):
        pltpu.sync_copy(zb, acc.at[pl.ds(sid * WPS + i * ZB, ZB)])
    plsc.subcore_barrier()
    cdescs = []
    for c in range(NCHS):
        cdescs.append(pltpu.async_copy(
            ones, acc.at[labs.at[c]], ssem[c % 2], add=True,
        ))
    for d in cdescs:
        d.wait()
    plsc.subcore_barrier()
    pltpu.sync_copy(
        acc.at[pl.ds(sid * WPS, WPS)], cp_hbm.at[cid, pl.ds(sid * WPS, WPS)]
    )


def _avg_body(fp_ref, cp_ref, out_ref):
    s = fp_ref[0] + fp_ref[1]
    c = cp_ref[0, :, 0:1] + cp_ref[1, :, 0:1]
    out_ref[...] = s / c


def _combine(fp, cp):
    mb = 512
    return pl.pallas_call(
        _avg_body,
        grid=(NSEG_PAD // mb,),
        in_specs=[
            pl.BlockSpec((NC, mb, NB), lambda i: (0, i, 0)),
            pl.BlockSpec((NC, mb, NB), lambda i: (0, i, 0)),
        ],
        out_specs=pl.BlockSpec((mb, NB), lambda i: (i, 0)),
        out_shape=jax.ShapeDtypeStruct((NSEG_PAD, NB), jnp.float32),
    )(fp, cp)


def kernel(x0, x1, x2, t_feat, conv_w, conv_b):
    x0f = x0.astype(jnp.int32).reshape(-1)
    x1g = x1.astype(jnp.int32).reshape(NW, NCHS, R)
    w2 = jnp.transpose(conv_w, (1, 2, 0)).reshape(FEAT, WIN * NB)
    bias_full = jnp.concatenate(
        [conv_b, jnp.zeros(((WIN - 1) * NB,), jnp.float32)]
    ).reshape(1, WIN * NB)

    p = _project(t_feat, w2, bias_full)
    p2 = p.reshape(N_NODES * WIN, NB)
    win_enc = _winenc(p2, x0f)
    fp, cp = _segsum(win_enc, x1g)
    favg_pad = _combine(fp, cp)
    return (win_enc, favg_pad[:N_NODES])


# speedup vs baseline: 6.7168x; 1.0043x over previous
import functools

import jax
import jax.numpy as jnp
from jax import lax
from jax.experimental import pallas as pl
from jax.experimental.pallas import tpu as pltpu
from jax.experimental.pallas import tpu_sc as plsc

N_CTX = 64000
WIN = 10
N_NODES = 10000
FEAT = 128
NB = 128
DROP = 0.5

NC = 2
NS = 16
NW = NC * NS
CPW = N_CTX // NW

CH = 40
NCHUNK = CPW // CH
ROWS = CH * WIN
GN = 5
GR = ROWS // GN

R = 80
NCHS = CPW // R
NSEG_PAD = 10240
NPASS = 1
NSEG_H = NSEG_PAD // NPASS
ACC_ROWS = NSEG_H + NS * 8
WPS = NSEG_H // NS
ZB = 64

_MESH = plsc.VectorSubcoreMesh(
    core_axis_name="c", subcore_axis_name="s", num_cores=NC, num_subcores=NS
)


def _proj_body(tf_ref, w2_ref, bias_ref, out_ref):
    acc = lax.dot_general(
        tf_ref[...], w2_ref[...], (((1,), (0,)), ((), ())),
        preferred_element_type=jnp.float32,
        precision=lax.Precision.HIGHEST,
    )
    out_ref[...] = acc * (1.0 - DROP) + bias_ref[...]


def _project(t_feat, w2, bias_full):
    mb = 400
    return pl.pallas_call(
        _proj_body,
        grid=(N_NODES // mb,),
        in_specs=[
            pl.BlockSpec((mb, FEAT), lambda i: (i, 0)),
            pl.BlockSpec((FEAT, WIN * NB), lambda i: (0, 0)),
            pl.BlockSpec((1, WIN * NB), lambda i: (0, 0)),
        ],
        out_specs=pl.BlockSpec((mb, WIN * NB), lambda i: (i, 0)),
        out_shape=jax.ShapeDtypeStruct((N_NODES, WIN * NB), jnp.float32),
    )(t_feat, w2, bias_full)


@functools.partial(
    pl.kernel,
    out_type=jax.ShapeDtypeStruct((N_CTX, NB), jnp.float32),
    mesh=_MESH,
    scratch_types=[
        pltpu.VMEM((ROWS,), jnp.int32),
        pltpu.VMEM((ROWS,), jnp.int32),
        pltpu.VMEM((GN, GR), jnp.int32),
        pltpu.VMEM((GN, GR), jnp.int32),
        pltpu.VMEM((ROWS, NB), jnp.float32),
        pltpu.VMEM((ROWS, NB), jnp.float32),
        pltpu.VMEM((CH, NB), jnp.float32),
        pltpu.SemaphoreType.DMA,
        pltpu.SemaphoreType.DMA,
    ],
)
def _winenc(p2_hbm, x0_hbm, out_hbm, x0c0, x0c1, idxc0, idxc1,
            rows0, rows1, outc, gs0, gs1):
    wid = lax.axis_index("s") * NC + lax.axis_index("c")
    lanes = lax.iota(jnp.int32, 16)
    gsem = (gs0, gs1)
    x0cs = (x0c0, x0c1)
    idxcs = (idxc0, idxc1)
    rowss = (rows0, rows1)

    def _prep(c, bi):
        x0c, idxc, rows = x0cs[bi], idxcs[bi], rowss[bi]
        base_j = (wid * CPW + c * CH) * WIN
        pltpu.sync_copy(x0_hbm.at[pl.ds(base_j, ROWS)], x0c)
        for k in range(ROWS // 16):
            x = x0c[pl.ds(k * 16, 16)]
            wpos = lax.rem(lanes + (k * 16) % WIN, WIN)
            idxc[k // GN, pl.ds((k % GN) * 16, 16)] = x * WIN + wpos
        for g in range(GN):
            pltpu.async_copy(
                p2_hbm.at[idxc.at[g]],
                rows.at[pl.ds(g * GR, GR)], gsem[bi],
            )

    def _drain(bi):
        pltpu.make_async_copy(
            p2_hbm.at[pl.ds(0, ROWS)], rowss[bi], gsem[bi]
        ).wait()

    def _compute(c, bi):
        rows = rowss[bi]

        def ctx_body(b, carry2):
            r0 = b * WIN
            for h in range(NB // 16):
                sl = pl.ds(h * 16, 16)
                acc = rows[r0, sl]
                for w in range(1, WIN):
                    acc = acc + rows[r0 + w, sl]
                outc[b, sl] = acc
            return carry2

        lax.fori_loop(0, CH, ctx_body, 0)
        pltpu.sync_copy(outc, out_hbm.at[pl.ds(wid * CPW + c * CH, CH)])

    _prep(0, 0)

    def pair_body(p, carry):
        c0 = 2 * p
        _prep(c0 + 1, 1)
        _drain(0)
        _compute(c0, 0)
        _prep(lax.rem(c0 + 2, NCHUNK), 0)
        _drain(1)
        _compute(c0 + 1, 1)
        return carry

    lax.fori_loop(0, NCHUNK // 2, pair_body, 0)
    _drain(0)


R = 80
NCHS = CPW // R
GCH = 1
NG = NCHS
GROWS = R
NSEG_PAD = 10240
WPS = NSEG_PAD // NS
ZB = 64


@functools.partial(
    pl.kernel,
    out_type=(
        jax.ShapeDtypeStruct((NC, NSEG_PAD, NB), jnp.float32),
        jax.ShapeDtypeStruct((NC, NSEG_PAD, NB), jnp.float32),
    ),
    mesh=_MESH,
    scratch_types=[
        pltpu.VMEM((2, GROWS, NB), jnp.float32),
        pltpu.VMEM((NCHS, R), jnp.int32),
        pltpu.VMEM((R, NB), jnp.float32),
        pltpu.VMEM((ZB, NB), jnp.float32),
        pltpu.VMEM_SHARED((NSEG_PAD, NB), jnp.float32),
        pltpu.SemaphoreType.DMA,
        pltpu.SemaphoreType.DMA,
        pltpu.SemaphoreType.DMA,
        pltpu.SemaphoreType.DMA,
    ],
)
def _segsum(win_hbm, x1g_hbm, fp_hbm, cp_hbm, rowg, labs, ones, zb,
            acc, rs0, rs1, ss0, ss1):
    cid = lax.axis_index("c")
    sid = lax.axis_index("s")
    wid = sid * NC + cid
    zero16 = jnp.zeros((16,), jnp.float32)
    one16 = jnp.ones((16,), jnp.float32)
    rsem = (rs0, rs1)
    ssem = (ss0, ss1)

    def fill_body(i, carry):
        for h in range(NB // 16):
            zb[i, pl.ds(h * 16, 16)] = zero16
            ones[i, pl.ds(h * 16, 16)] = one16
        return carry

    lax.fori_loop(0, ZB, fill_body, 0)

    def ones_body(i, carry):
        for h in range(NB // 16):
            ones[ZB + i, pl.ds(h * 16, 16)] = one16
        return carry

    lax.fori_loop(0, R - ZB, ones_body, 0)

    pltpu.sync_copy(x1g_hbm.at[wid], labs)
    for i in range(WPS // ZB):
        pltpu.sync_copy(zb, acc.at[pl.ds(sid * WPS + i * ZB, ZB)])
    plsc.subcore_barrier()

    row0 = wid * CPW
    pend = [None, None]
    rcps = [None, None]
    rcps[0] = pltpu.async_copy(
        win_hbm.at[pl.ds(row0, R)], rowg.at[0], rsem[0]
    )
    for c in range(NCHS):
        bi = c % 2
        if pend[1 - bi] is not None:
            pend[1 - bi].wait()
            pend[1 - bi] = None
        if c + 1 < NCHS:
            rcps[1 - bi] = pltpu.async_copy(
                win_hbm.at[pl.ds(row0 + (c + 1) * R, R)],
                rowg.at[1 - bi], rsem[1 - bi],
            )
        rcps[bi].wait()
        pend[bi] = pltpu.async_copy(
            rowg.at[bi], acc.at[labs.at[c]], ssem[bi], add=True,
        )
    for bi in range(2):
        if pend[bi] is not None:
            pend[bi].wait()
    plsc.subcore_barrier()

    pltpu.sync_copy(
        acc.at[pl.ds(sid * WPS, WPS)], fp_hbm.at[cid, pl.ds(sid * WPS, WPS)]
    )
    plsc.subcore_barrier()

    for i in range(WPS // ZB):
        pltpu.sync_copy(zb, acc.at[pl.ds(sid * WPS + i * ZB, ZB)])
    plsc.subcore_barrier()
    cdescs = []
    for c in range(NCHS):
        cdescs.append(pltpu.async_copy(
            ones, acc.at[labs.at[c]], ssem[c % 2], add=True,
        ))
    for d in cdescs:
        d.wait()
    plsc.subcore_barrier()
    pltpu.sync_copy(
        acc.at[pl.ds(sid * WPS, WPS)], cp_hbm.at[cid, pl.ds(sid * WPS, WPS)]
    )


def _avg_body(fp_ref, cp_ref, out_ref):
    s = fp_ref[0] + fp_ref[1]
    c = cp_ref[0, :, 0:1] + cp_ref[1, :, 0:1]
    out_ref[...] = s / c


def _combine(fp, cp):
    mb = 400
    return pl.pallas_call(
        _avg_body,
        grid=(N_NODES // mb,),
        in_specs=[
            pl.BlockSpec((NC, mb, NB), lambda i: (0, i, 0)),
            pl.BlockSpec((NC, mb, NB), lambda i: (0, i, 0)),
        ],
        out_specs=pl.BlockSpec((mb, NB), lambda i: (i, 0)),
        out_shape=jax.ShapeDtypeStruct((N_NODES, NB), jnp.float32),
    )(fp, cp)


def kernel(x0, x1, x2, t_feat, conv_w, conv_b):
    x0f = x0.astype(jnp.int32).reshape(-1)
    x1g = x1.astype(jnp.int32).reshape(NW, NCHS, R)
    w2 = jnp.transpose(conv_w, (1, 2, 0)).reshape(FEAT, WIN * NB)
    bias_full = jnp.concatenate(
        [conv_b, jnp.zeros(((WIN - 1) * NB,), jnp.float32)]
    ).reshape(1, WIN * NB)

    p = _project(t_feat, w2, bias_full)
    p2 = p.reshape(N_NODES * WIN, NB)
    win_enc = _winenc(p2, x0f)
    fp, cp = _segsum(win_enc, x1g)
    feat_avg = _combine(fp, cp)
    return (win_enc, feat_avg)


# speedup vs baseline: 7.1165x; 1.0595x over previous
import functools

import jax
import jax.numpy as jnp
from jax import lax
from jax.experimental import pallas as pl
from jax.experimental.pallas import tpu as pltpu
from jax.experimental.pallas import tpu_sc as plsc

N_CTX = 64000
WIN = 10
N_NODES = 10000
FEAT = 128
NB = 128
DROP = 0.5

NC = 2
NS = 16
NW = NC * NS
CPW = N_CTX // NW

CH = 40
NCHUNK = CPW // CH
ROWS = CH * WIN
GN = 5
GR = ROWS // GN

R = 80
NCHS = CPW // R
NSEG_PAD = 10240
NPASS = 1
NSEG_H = NSEG_PAD // NPASS
ACC_ROWS = NSEG_H + NS * 8
WPS = NSEG_H // NS
ZB = 64

_MESH = plsc.VectorSubcoreMesh(
    core_axis_name="c", subcore_axis_name="s", num_cores=NC, num_subcores=NS
)


def _proj_body(tf_ref, w2_ref, bias_ref, out_ref):
    acc = lax.dot_general(
        tf_ref[...], w2_ref[...], (((1,), (0,)), ((), ())),
        preferred_element_type=jnp.float32,
        precision=lax.Precision.HIGHEST,
    )
    out_ref[...] = acc * (1.0 - DROP) + bias_ref[...]


def _project(t_feat, w2, bias_full):
    mb = 400
    return pl.pallas_call(
        _proj_body,
        grid=(N_NODES // mb,),
        in_specs=[
            pl.BlockSpec((mb, FEAT), lambda i: (i, 0)),
            pl.BlockSpec((FEAT, WIN * NB), lambda i: (0, 0)),
            pl.BlockSpec((1, WIN * NB), lambda i: (0, 0)),
        ],
        out_specs=pl.BlockSpec((mb, WIN * NB), lambda i: (i, 0)),
        out_shape=jax.ShapeDtypeStruct((N_NODES, WIN * NB), jnp.float32),
    )(t_feat, w2, bias_full)


@functools.partial(
    pl.kernel,
    out_type=jax.ShapeDtypeStruct((N_CTX, NB), jnp.float32),
    mesh=_MESH,
    scratch_types=[
        pltpu.VMEM((ROWS,), jnp.int32),
        pltpu.VMEM((ROWS,), jnp.int32),
        pltpu.VMEM((GN, GR), jnp.int32),
        pltpu.VMEM((GN, GR), jnp.int32),
        pltpu.VMEM((ROWS, NB), jnp.float32),
        pltpu.VMEM((ROWS, NB), jnp.float32),
        pltpu.VMEM((CH, NB), jnp.float32),
        pltpu.VMEM((CH, NB), jnp.float32),
        pltpu.SemaphoreType.DMA,
        pltpu.SemaphoreType.DMA,
        pltpu.SemaphoreType.DMA,
        pltpu.SemaphoreType.DMA,
    ],
)
def _winenc(p2_hbm, x0_hbm, out_hbm, x0c0, x0c1, idxc0, idxc1,
            rows0, rows1, outc0, outc1, gs0, gs1, xs0, xs1):
    wid = lax.axis_index("s") * NC + lax.axis_index("c")
    lanes = lax.iota(jnp.int32, 16)
    gsem = (gs0, gs1)
    xsem = (xs0, xs1)
    x0cs = (x0c0, x0c1)
    idxcs = (idxc0, idxc1)
    rowss = (rows0, rows1)
    outcs = (outc0, outc1)

    def _fire_x0(c, bi):
        base_j = (wid * CPW + c * CH) * WIN
        pltpu.async_copy(x0_hbm.at[pl.ds(base_j, ROWS)], x0cs[bi], xsem[bi])

    def _build_fire(bi):
        x0c, idxc, rows = x0cs[bi], idxcs[bi], rowss[bi]
        pltpu.make_async_copy(
            x0_hbm.at[pl.ds(0, ROWS)], x0c, xsem[bi]
        ).wait()
        for k in range(ROWS // 16):
            x = x0c[pl.ds(k * 16, 16)]
            wpos = lax.rem(lanes + (k * 16) % WIN, WIN)
            idxc[k // GN, pl.ds((k % GN) * 16, 16)] = x * WIN + wpos
        for g in range(GN):
            pltpu.async_copy(
                p2_hbm.at[idxc.at[g]],
                rows.at[pl.ds(g * GR, GR)], gsem[bi],
            )

    def _drain_rows(bi):
        pltpu.make_async_copy(
            p2_hbm.at[pl.ds(0, ROWS)], rowss[bi], gsem[bi]
        ).wait()

    def _compute(c, bi):
        rows, outc = rowss[bi], outcs[bi]

        def ctx_body(b, carry2):
            r0 = b * WIN
            for h in range(NB // 16):
                sl = pl.ds(h * 16, 16)
                acc = rows[r0, sl]
                for w in range(1, WIN):
                    acc = acc + rows[r0 + w, sl]
                outc[b, sl] = acc
            return carry2

        lax.fori_loop(0, CH, ctx_body, 0)
        pltpu.sync_copy(outc, out_hbm.at[pl.ds(wid * CPW + c * CH, CH)])

    _fire_x0(0, 0)
    _build_fire(0)
    _fire_x0(1, 1)

    def pair_body(p, carry):
        c0 = 2 * p
        _build_fire(1)
        _fire_x0(lax.rem(c0 + 2, NCHUNK), 0)
        _drain_rows(0)
        _compute(c0, 0)
        _build_fire(0)
        _fire_x0(lax.rem(c0 + 3, NCHUNK), 1)
        _drain_rows(1)
        _compute(c0 + 1, 1)
        return carry

    lax.fori_loop(0, NCHUNK // 2, pair_body, 0)
    _drain_rows(0)
    pltpu.make_async_copy(
        x0_hbm.at[pl.ds(0, ROWS)], x0cs[1], xsem[1]
    ).wait()


R = 80
NCHS = CPW // R
GCH = 1
NG = NCHS
GROWS = R
NSEG_PAD = 10240
WPS = NSEG_PAD // NS
ZB = 64


@functools.partial(
    pl.kernel,
    out_type=(
        jax.ShapeDtypeStruct((NC, NSEG_PAD, NB), jnp.float32),
        jax.ShapeDtypeStruct((NC, NSEG_PAD, NB), jnp.float32),
    ),
    mesh=_MESH,
    scratch_types=[
        pltpu.VMEM((2, GROWS, NB), jnp.float32),
        pltpu.VMEM((NCHS, R), jnp.int32),
        pltpu.VMEM((R, NB), jnp.float32),
        pltpu.VMEM((ZB, NB), jnp.float32),
        pltpu.VMEM_SHARED((NSEG_PAD, NB), jnp.float32),
        pltpu.SemaphoreType.DMA,
        pltpu.SemaphoreType.DMA,
        pltpu.SemaphoreType.DMA,
        pltpu.SemaphoreType.DMA,
    ],
)
def _segsum(win_hbm, x1g_hbm, fp_hbm, cp_hbm, rowg, labs, ones, zb,
            acc, rs0, rs1, ss0, ss1):
    cid = lax.axis_index("c")
    sid = lax.axis_index("s")
    wid = sid * NC + cid
    zero16 = jnp.zeros((16,), jnp.float32)
    one16 = jnp.ones((16,), jnp.float32)
    rsem = (rs0, rs1)
    ssem = (ss0, ss1)

    def fill_body(i, carry):
        for h in range(NB // 16):
            zb[i, pl.ds(h * 16, 16)] = zero16
            ones[i, pl.ds(h * 16, 16)] = one16
        return carry

    lax.fori_loop(0, ZB, fill_body, 0)

    def ones_body(i, carry):
        for h in range(NB // 16):
            ones[ZB + i, pl.ds(h * 16, 16)] = one16
        return carry

    lax.fori_loop(0, R - ZB, ones_body, 0)

    pltpu.sync_copy(x1g_hbm.at[wid], labs)
    for i in range(WPS // ZB):
        pltpu.sync_copy(zb, acc.at[pl.ds(sid * WPS + i * ZB, ZB)])
    plsc.subcore_barrier()

    row0 = wid * CPW
    pend = [None, None]
    rcps = [None, None]
    rcps[0] = pltpu.async_copy(
        win_hbm.at[pl.ds(row0, R)], rowg.at[0], rsem[0]
    )
    for c in range(NCHS):
        bi = c % 2
        if pend[1 - bi] is not None:
            pend[1 - bi].wait()
            pend[1 - bi] = None
        if c + 1 < NCHS:
            rcps[1 - bi] = pltpu.async_copy(
                win_hbm.at[pl.ds(row0 + (c + 1) * R, R)],
                rowg.at[1 - bi], rsem[1 - bi],
            )
        rcps[bi].wait()
        pend[bi] = pltpu.async_copy(
            rowg.at[bi], acc.at[labs.at[c]], ssem[bi], add=True,
        )
    for bi in range(2):
        if pend[bi] is not None:
            pend[bi].wait()
    plsc.subcore_barrier()

    pltpu.sync_copy(
        acc.at[pl.ds(sid * WPS, WPS)], fp_hbm.at[cid, pl.ds(sid * WPS, WPS)]
    )
    plsc.subcore_barrier()

    for i in range(WPS // ZB):
        pltpu.sync_copy(zb, acc.at[pl.ds(sid * WPS + i * ZB, ZB)])
    plsc.subcore_barrier()
    cdescs = []
    for c in range(NCHS):
        cdescs.append(pltpu.async_copy(
            ones, acc.at[labs.at[c]], ssem[c % 2], add=True,
        ))
    for d in cdescs:
        d.wait()
    plsc.subcore_barrier()
    pltpu.sync_copy(
        acc.at[pl.ds(sid * WPS, WPS)], cp_hbm.at[cid, pl.ds(sid * WPS, WPS)]
    )


def _avg_body(fp_ref, cp_ref, out_ref):
    s = fp_ref[0] + fp_ref[1]
    c = cp_ref[0, :, 0:1] + cp_ref[1, :, 0:1]
    out_ref[...] = s / c


def _combine(fp, cp):
    mb = 400
    return pl.pallas_call(
        _avg_body,
        grid=(N_NODES // mb,),
        in_specs=[
            pl.BlockSpec((NC, mb, NB), lambda i: (0, i, 0)),
            pl.BlockSpec((NC, mb, NB), lambda i: (0, i, 0)),
        ],
        out_specs=pl.BlockSpec((mb, NB), lambda i: (i, 0)),
        out_shape=jax.ShapeDtypeStruct((N_NODES, NB), jnp.float32),
    )(fp, cp)


def kernel(x0, x1, x2, t_feat, conv_w, conv_b):
    x0f = x0.astype(jnp.int32).reshape(-1)
    x1g = x1.astype(jnp.int32).reshape(NW, NCHS, R)
    w2 = jnp.transpose(conv_w, (1, 2, 0)).reshape(FEAT, WIN * NB)
    bias_full = jnp.concatenate(
        [conv_b, jnp.zeros(((WIN - 1) * NB,), jnp.float32)]
    ).reshape(1, WIN * NB)

    p = _project(t_feat, w2, bias_full)
    p2 = p.reshape(N_NODES * WIN, NB)
    win_enc = _winenc(p2, x0f)
    fp, cp = _segsum(win_enc, x1g)
    feat_avg = _combine(fp, cp)
    return (win_enc, feat_avg)


# speedup vs baseline: 7.4217x; 1.0429x over previous
import functools

import jax
import jax.numpy as jnp
from jax import lax
from jax.experimental import pallas as pl
from jax.experimental.pallas import tpu as pltpu
from jax.experimental.pallas import tpu_sc as plsc

N_CTX = 64000
WIN = 10
N_NODES = 10000
FEAT = 128
NB = 128
DROP = 0.5

NC = 2
NS = 16
NW = NC * NS
CPW = N_CTX // NW

CH = 40
NCHUNK = CPW // CH
ROWS = CH * WIN
GN = 5
GR = ROWS // GN

R = 80
NCHS = CPW // R
NSEG_PAD = 10240
NPASS = 1
NSEG_H = NSEG_PAD // NPASS
ACC_ROWS = NSEG_H + NS * 8
WPS = NSEG_H // NS
ZB = 64

_MESH = plsc.VectorSubcoreMesh(
    core_axis_name="c", subcore_axis_name="s", num_cores=NC, num_subcores=NS
)


def _proj_body(tf_ref, w2_ref, bias_ref, out_ref):
    acc = lax.dot_general(
        tf_ref[...], w2_ref[...], (((1,), (0,)), ((), ())),
        preferred_element_type=jnp.float32,
    )
    out_ref[...] = acc * (1.0 - DROP) + bias_ref[...]


def _project(t_feat, w2, bias_full):
    mb = 400
    return pl.pallas_call(
        _proj_body,
        grid=(N_NODES // mb,),
        in_specs=[
            pl.BlockSpec((mb, FEAT), lambda i: (i, 0)),
            pl.BlockSpec((FEAT, WIN * NB), lambda i: (0, 0)),
            pl.BlockSpec((1, WIN * NB), lambda i: (0, 0)),
        ],
        out_specs=pl.BlockSpec((mb, WIN * NB), lambda i: (i, 0)),
        out_shape=jax.ShapeDtypeStruct((N_NODES, WIN * NB), jnp.float32),
    )(t_feat, w2, bias_full)


@functools.partial(
    pl.kernel,
    out_type=jax.ShapeDtypeStruct((N_CTX, NB), jnp.float32),
    mesh=_MESH,
    scratch_types=[
        pltpu.VMEM((ROWS,), jnp.int32),
        pltpu.VMEM((ROWS,), jnp.int32),
        pltpu.VMEM((GN, GR), jnp.int32),
        pltpu.VMEM((GN, GR), jnp.int32),
        pltpu.VMEM((ROWS, NB), jnp.float32),
        pltpu.VMEM((ROWS, NB), jnp.float32),
        pltpu.VMEM((CH, NB), jnp.float32),
        pltpu.VMEM((CH, NB), jnp.float32),
        pltpu.SemaphoreType.DMA,
        pltpu.SemaphoreType.DMA,
        pltpu.SemaphoreType.DMA,
        pltpu.SemaphoreType.DMA,
    ],
)
def _winenc(p2_hbm, x0_hbm, out_hbm, x0c0, x0c1, idxc0, idxc1,
            rows0, rows1, outc0, outc1, gs0, gs1, xs0, xs1):
    wid = lax.axis_index("s") * NC + lax.axis_index("c")
    lanes = lax.iota(jnp.int32, 16)
    gsem = (gs0, gs1)
    xsem = (xs0, xs1)
    x0cs = (x0c0, x0c1)
    idxcs = (idxc0, idxc1)
    rowss = (rows0, rows1)
    outcs = (outc0, outc1)

    def _fire_x0(c, bi):
        base_j = (wid * CPW + c * CH) * WIN
        pltpu.async_copy(x0_hbm.at[pl.ds(base_j, ROWS)], x0cs[bi], xsem[bi])

    def _build_fire(bi):
        x0c, idxc, rows = x0cs[bi], idxcs[bi], rowss[bi]
        pltpu.make_async_copy(
            x0_hbm.at[pl.ds(0, ROWS)], x0c, xsem[bi]
        ).wait()
        for k in range(ROWS // 16):
            x = x0c[pl.ds(k * 16, 16)]
            wpos = lax.rem(lanes + (k * 16) % WIN, WIN)
            idxc[k // GN, pl.ds((k % GN) * 16, 16)] = x * WIN + wpos
        for g in range(GN):
            pltpu.async_copy(
                p2_hbm.at[idxc.at[g]],
                rows.at[pl.ds(g * GR, GR)], gsem[bi],
            )

    def _drain_rows(bi):
        pltpu.make_async_copy(
            p2_hbm.at[pl.ds(0, ROWS)], rowss[bi], gsem[bi]
        ).wait()

    def _compute(c, bi):
        rows, outc = rowss[bi], outcs[bi]

        def ctx_body(b, carry2):
            r0 = b * WIN
            for h in range(NB // 16):
                sl = pl.ds(h * 16, 16)
                acc = rows[r0, sl]
                for w in range(1, WIN):
                    acc = acc + rows[r0 + w, sl]
                outc[b, sl] = acc
            return carry2

        lax.fori_loop(0, CH, ctx_body, 0)
        pltpu.sync_copy(outc, out_hbm.at[pl.ds(wid * CPW + c * CH, CH)])

    _fire_x0(0, 0)
    _build_fire(0)
    _fire_x0(1, 1)

    def pair_body(p, carry):
        c0 = 2 * p
        _build_fire(1)
        _fire_x0(lax.rem(c0 + 2, NCHUNK), 0)
        _drain_rows(0)
        _compute(c0, 0)
        _build_fire(0)
        _fire_x0(lax.rem(c0 + 3, NCHUNK), 1)
        _drain_rows(1)
        _compute(c0 + 1, 1)
        return carry

    lax.fori_loop(0, NCHUNK // 2, pair_body, 0)
    _drain_rows(0)
    pltpu.make_async_copy(
        x0_hbm.at[pl.ds(0, ROWS)], x0cs[1], xsem[1]
    ).wait()


R = 80
NCHS = CPW // R
GCH = 1
NG = NCHS
GROWS = R
NSEG_PAD = 10240
WPS = NSEG_PAD // NS
ZB = 64


@functools.partial(
    pl.kernel,
    out_type=(
        jax.ShapeDtypeStruct((NC, NSEG_PAD, NB), jnp.float32),
        jax.ShapeDtypeStruct((NC, NSEG_PAD, NB), jnp.float32),
    ),
    mesh=_MESH,
    scratch_types=[
        pltpu.VMEM((2, GROWS, NB), jnp.float32),
        pltpu.VMEM((NCHS, R), jnp.int32),
        pltpu.VMEM((R, NB), jnp.float32),
        pltpu.VMEM((ZB, NB), jnp.float32),
        pltpu.VMEM_SHARED((NSEG_PAD, NB), jnp.float32),
        pltpu.SemaphoreType.DMA,
        pltpu.SemaphoreType.DMA,
        pltpu.SemaphoreType.DMA,
        pltpu.SemaphoreType.DMA,
    ],
)
def _segsum(win_hbm, x1g_hbm, fp_hbm, cp_hbm, rowg, labs, ones, zb,
            acc, rs0, rs1, ss0, ss1):
    cid = lax.axis_index("c")
    sid = lax.axis_index("s")
    wid = sid * NC + cid
    zero16 = jnp.zeros((16,), jnp.float32)
    one16 = jnp.ones((16,), jnp.float32)
    rsem = (rs0, rs1)
    ssem = (ss0, ss1)

    def fill_body(i, carry):
        for h in range(NB // 16):
            zb[i, pl.ds(h * 16, 16)] = zero16
            ones[i, pl.ds(h * 16, 16)] = one16
        return carry

    lax.fori_loop(0, ZB, fill_body, 0)

    def ones_body(i, carry):
        for h in range(NB // 16):
            ones[ZB + i, pl.ds(h * 16, 16)] = one16
        return carry

    lax.fori_loop(0, R - ZB, ones_body, 0)

    pltpu.sync_copy(x1g_hbm.at[wid], labs)
    for i in range(WPS // ZB):
        pltpu.sync_copy(zb, acc.at[pl.ds(sid * WPS + i * ZB, ZB)])
    plsc.subcore_barrier()

    row0 = wid * CPW
    pend = [None, None]
    rcps = [None, None]
    rcps[0] = pltpu.async_copy(
        win_hbm.at[pl.ds(row0, R)], rowg.at[0], rsem[0]
    )
    for c in range(NCHS):
        bi = c % 2
        if pend[1 - bi] is not None:
            pend[1 - bi].wait()
            pend[1 - bi] = None
        if c + 1 < NCHS:
            rcps[1 - bi] = pltpu.async_copy(
                win_hbm.at[pl.ds(row0 + (c + 1) * R, R)],
                rowg.at[1 - bi], rsem[1 - bi],
            )
        rcps[bi].wait()
        pend[bi] = pltpu.async_copy(
            rowg.at[bi], acc.at[labs.at[c]], ssem[bi], add=True,
        )
    for bi in range(2):
        if pend[bi] is not None:
            pend[bi].wait()
    plsc.subcore_barrier()

    pltpu.sync_copy(
        acc.at[pl.ds(sid * WPS, WPS)], fp_hbm.at[cid, pl.ds(sid * WPS, WPS)]
    )
    plsc.subcore_barrier()

    for i in range(WPS // ZB):
        pltpu.sync_copy(zb, acc.at[pl.ds(sid * WPS + i * ZB, ZB)])
    plsc.subcore_barrier()
    cdescs = []
    for c in range(NCHS):
        cdescs.append(pltpu.async_copy(
            ones, acc.at[labs.at[c]], ssem[c % 2], add=True,
        ))
    for d in cdescs:
        d.wait()
    plsc.subcore_barrier()
    pltpu.sync_copy(
        acc.at[pl.ds(sid * WPS, WPS)], cp_hbm.at[cid, pl.ds(sid * WPS, WPS)]
    )


def _avg_body(fp_ref, cp_ref, out_ref):
    s = fp_ref[0] + fp_ref[1]
    c = cp_ref[0, :, 0:1] + cp_ref[1, :, 0:1]
    out_ref[...] = s / c


def _combine(fp, cp):
    mb = 400
    return pl.pallas_call(
        _avg_body,
        grid=(N_NODES // mb,),
        in_specs=[
            pl.BlockSpec((NC, mb, NB), lambda i: (0, i, 0)),
            pl.BlockSpec((NC, mb, NB), lambda i: (0, i, 0)),
        ],
        out_specs=pl.BlockSpec((mb, NB), lambda i: (i, 0)),
        out_shape=jax.ShapeDtypeStruct((N_NODES, NB), jnp.float32),
    )(fp, cp)


def kernel(x0, x1, x2, t_feat, conv_w, conv_b):
    x0f = x0.astype(jnp.int32).reshape(-1)
    x1g = x1.astype(jnp.int32).reshape(NW, NCHS, R)
    w2 = jnp.transpose(conv_w, (1, 2, 0)).reshape(FEAT, WIN * NB)
    bias_full = jnp.concatenate(
        [conv_b, jnp.zeros(((WIN - 1) * NB,), jnp.float32)]
    ).reshape(1, WIN * NB)

    p = _project(t_feat, w2, bias_full)
    p2 = p.reshape(N_NODES * WIN, NB)
    win_enc = _winenc(p2, x0f)
    fp, cp = _segsum(win_enc, x1g)
    feat_avg = _combine(fp, cp)
    return (win_enc, feat_avg)
